# Initial kernel scaffold; baseline (speedup 1.0000x reference)
import jax
import jax.numpy as jnp
from jax.experimental import pallas as pl


def kernel(x, edge_index, W1, b1, W2, b2, fc1W, fc1b, fc2W, fc2b, fc3W, fc3b):
    raise NotImplementedError("write your pallas kernel here")


# speedup vs baseline: 7.5525x; 7.5525x over previous
import functools

import jax
import jax.numpy as jnp
from jax import lax
from jax.experimental import pallas as pl
from jax.experimental.pallas import tpu as pltpu
from jax.experimental.pallas import tpu_sc as plsc

N_NODES = 10000
N_EDGES = 320000
D = 128

NC = 2
NS = 16
NW = NC * NS
N_PAD = 10240
RPT = N_PAD // NS
EPT = N_EDGES // NW
CH = 80
NCHUNK = EPT // CH

_mesh = plsc.VectorSubcoreMesh(core_axis_name="c", subcore_axis_name="s")


def _leaky(x):
    return jnp.where(x >= 0, x, 0.01 * x)


def _sc_degree_body(dst_hbm, zeros_hbm, out_hbm, acc, idxb, onesb):
    c = lax.axis_index("c")
    s = lax.axis_index("s")
    wid = c * NS + s
    r0 = s * RPT
    pltpu.sync_copy(zeros_hbm.at[pl.ds(r0, RPT)], acc.at[pl.ds(r0, RPT)])
    for i in range(CH // 16):
        onesb[pl.ds(i * 16, 16)] = jnp.ones((16,), jnp.float32)
    plsc.subcore_barrier()
    base0 = wid * EPT

    @pl.loop(0, NCHUNK)
    def _chunk(j):
        pltpu.sync_copy(dst_hbm.at[pl.ds(base0 + j * CH, CH)], idxb)
        pltpu.sync_copy(onesb, acc.at[idxb], add=True)

    plsc.subcore_barrier()
    pltpu.sync_copy(acc.at[pl.ds(r0, RPT)], out_hbm.at[c, pl.ds(r0, RPT)])


_SC_DEGREE_KW = dict(
    out_type=jax.ShapeDtypeStruct((NC, N_PAD), jnp.float32),
    mesh=_mesh,
    scratch_types=[
        pltpu.VMEM_SHARED((N_PAD,), jnp.float32),
        pltpu.VMEM((CH,), jnp.int32),
        pltpu.VMEM((CH,), jnp.float32),
    ],
)
_sc_degree = pl.kernel(**_SC_DEGREE_KW)(_sc_degree_body)


def _sc_conv_body(y_hbm, src_hbm, dst_hbm, init_hbm, out_hbm,
                  acc, srcb, dstb, rows, sem):
    c = lax.axis_index("c")
    s = lax.axis_index("s")
    wid = c * NS + s
    r0 = s * RPT
    pltpu.sync_copy(init_hbm.at[c, pl.ds(r0, RPT)], acc.at[pl.ds(r0, RPT)])
    plsc.subcore_barrier()
    base0 = wid * EPT

    @pl.loop(0, NCHUNK)
    def _chunk(j):
        base = base0 + j * CH
        pltpu.sync_copy(src_hbm.at[pl.ds(base, CH)], srcb)
        pltpu.sync_copy(dst_hbm.at[pl.ds(base, CH)], dstb)
        pltpu.async_copy(y_hbm.at[srcb], rows, sem).wait()
        pltpu.sync_copy(rows, acc.at[dstb], add=True)

    plsc.subcore_barrier()
    pltpu.sync_copy(acc.at[pl.ds(r0, RPT)], out_hbm.at[c, pl.ds(r0, RPT)])


_SC_CONV_KW = dict(
    out_type=jax.ShapeDtypeStruct((NC, N_PAD, D), jnp.float32),
    mesh=_mesh,
    scratch_types=[
        pltpu.VMEM_SHARED((N_PAD, D), jnp.float32),
        pltpu.VMEM((CH,), jnp.int32),
        pltpu.VMEM((CH,), jnp.int32),
        pltpu.VMEM((CH, D), jnp.float32),
        pltpu.SemaphoreType.DMA,
    ],
)
_sc_conv = pl.kernel(**_SC_CONV_KW, name="sc_conv1")(_sc_conv_body)
_sc_conv2 = pl.kernel(**_SC_CONV_KW, name="sc_conv2")(_sc_conv_body)


def _sc_edge_gather_body(a_hbm, b_hbm, src_hbm, dst_hbm, out_hbm,
                         srcb, dstb, bufa, bufb, sema, semb):
    c = lax.axis_index("c")
    s = lax.axis_index("s")
    wid = c * NS + s
    base0 = wid * EPT

    @pl.loop(0, NCHUNK)
    def _chunk(j):
        base = base0 + j * CH
        pltpu.sync_copy(src_hbm.at[pl.ds(base, CH)], srcb)
        pltpu.sync_copy(dst_hbm.at[pl.ds(base, CH)], dstb)
        cpa = pltpu.async_copy(a_hbm.at[srcb], bufa, sema)
        cpb = pltpu.async_copy(b_hbm.at[dstb], bufb, semb)
        cpa.wait()
        cpb.wait()

        @pl.loop(0, CH)
        def _row(i):
            for l in range(D // 16):
                sl = pl.ds(l * 16, 16)
                bufa[i, sl] = bufa[i, sl] + bufb[i, sl]

        pltpu.sync_copy(bufa, out_hbm.at[pl.ds(base, CH)])


_SC_EDGE_KW = dict(
    out_type=jax.ShapeDtypeStruct((N_EDGES, D), jnp.float32),
    mesh=_mesh,
    scratch_types=[
        pltpu.VMEM((CH,), jnp.int32),
        pltpu.VMEM((CH,), jnp.int32),
        pltpu.VMEM((CH, D), jnp.float32),
        pltpu.VMEM((CH, D), jnp.float32),
        pltpu.SemaphoreType.DMA,
        pltpu.SemaphoreType.DMA,
    ],
)
_sc_edge_gather = pl.kernel(**_SC_EDGE_KW)(_sc_edge_gather_body)


def _tc_pre_body(x_ref, w_ref, degp_ref, y_ref, dinv_ref):
    deg = degp_ref[0, :] + degp_ref[1, :] + 1.0
    dinv = lax.rsqrt(deg)[:, None]
    dinv_ref[...] = dinv
    xw = jnp.dot(x_ref[...], w_ref[...], preferred_element_type=jnp.float32)
    y_ref[...] = xw * dinv


def _tc_pre(x, w1, degp):
    blk = 640
    grid = N_PAD // blk
    return pl.pallas_call(
        _tc_pre_body,
        grid=(grid,),
        in_specs=[
            pl.BlockSpec((blk, D), lambda i: (i, 0)),
            pl.BlockSpec((D, D), lambda i: (0, 0)),
            pl.BlockSpec((NC, blk), lambda i: (0, i)),
        ],
        out_specs=[
            pl.BlockSpec((blk, D), lambda i: (i, 0)),
            pl.BlockSpec((blk, 1), lambda i: (i, 0)),
        ],
        out_shape=[
            jax.ShapeDtypeStruct((N_PAD, D), jnp.float32),
            jax.ShapeDtypeStruct((N_PAD, 1), jnp.float32),
        ],
    )(x, w1, degp)


def _tc_mid_body(sp_ref, dinv_ref, b_ref, w_ref, y2_ref):
    s = sp_ref[0] + sp_ref[1]
    dinv = dinv_ref[...]
    h = _leaky(s * dinv + b_ref[...][None, :])
    hw = jnp.dot(h, w_ref[...], preferred_element_type=jnp.float32)
    y2_ref[...] = hw * dinv


def _tc_mid(sp, dinv, b1, w2):
    blk = 640
    grid = N_PAD // blk
    return pl.pallas_call(
        _tc_mid_body,
        grid=(grid,),
        in_specs=[
            pl.BlockSpec((NC, blk, D), lambda i: (0, i, 0)),
            pl.BlockSpec((blk, 1), lambda i: (i, 0)),
            pl.BlockSpec((D,), lambda i: (0,)),
            pl.BlockSpec((D, D), lambda i: (0, 0)),
        ],
        out_specs=pl.BlockSpec((blk, D), lambda i: (i, 0)),
        out_shape=jax.ShapeDtypeStruct((N_PAD, D), jnp.float32),
    )(sp, dinv, b1, w2)


def _tc_post_body(sp_ref, dinv_ref, b_ref, fc1w_ref, fc1b_ref, a_ref, bb_ref):
    s = sp_ref[0] + sp_ref[1]
    dinv = dinv_ref[...]
    h = _leaky(s * dinv + b_ref[...][None, :])
    a_ref[...] = jnp.dot(h, fc1w_ref[: D, :],
                         preferred_element_type=jnp.float32)
    bb_ref[...] = jnp.dot(h, fc1w_ref[D:, :],
                          preferred_element_type=jnp.float32) + fc1b_ref[...][None, :]


def _tc_post(sp, dinv, b2, fc1w, fc1b):
    blk = 640
    grid = N_PAD // blk
    return pl.pallas_call(
        _tc_post_body,
        grid=(grid,),
        in_specs=[
            pl.BlockSpec((NC, blk, D), lambda i: (0, i, 0)),
            pl.BlockSpec((blk, 1), lambda i: (i, 0)),
            pl.BlockSpec((D,), lambda i: (0,)),
            pl.BlockSpec((2 * D, D), lambda i: (0, 0)),
            pl.BlockSpec((D,), lambda i: (0,)),
        ],
        out_specs=[
            pl.BlockSpec((blk, D), lambda i: (i, 0)),
            pl.BlockSpec((blk, D), lambda i: (i, 0)),
        ],
        out_shape=[
            jax.ShapeDtypeStruct((N_PAD, D), jnp.float32),
            jax.ShapeDtypeStruct((N_PAD, D), jnp.float32),
        ],
    )(sp, dinv, b2, fc1w, fc1b)


def _tc_mlp_body(e0_ref, w2_ref, b2_ref, w3_ref, b3_ref, out_ref):
    e1 = _leaky(e0_ref[...])
    e2 = _leaky(jnp.dot(e1, w2_ref[...], preferred_element_type=jnp.float32)
                + b2_ref[...][None, :])
    out_ref[...] = (jnp.dot(e2, w3_ref[...], preferred_element_type=jnp.float32)
                    + b3_ref[...][None, :])


def _tc_mlp(e0, fc2w, fc2b, fc3w, fc3b):
    blk = 2560
    grid = N_EDGES // blk
    return pl.pallas_call(
        _tc_mlp_body,
        grid=(grid,),
        in_specs=[
            pl.BlockSpec((blk, D), lambda i: (i, 0)),
            pl.BlockSpec((D, 64), lambda i: (0, 0)),
            pl.BlockSpec((64,), lambda i: (0,)),
            pl.BlockSpec((64, 3), lambda i: (0, 0)),
            pl.BlockSpec((3,), lambda i: (0,)),
        ],
        out_specs=pl.BlockSpec((blk, 3), lambda i: (i, 0)),
        out_shape=jax.ShapeDtypeStruct((N_EDGES, 3), jnp.float32),
    )(e0, fc2w, fc2b, fc3w, fc3b)


def kernel(x, edge_index, W1, b1, W2, b2, fc1W, fc1b, fc2W, fc2b, fc3W, fc3b):
    src = edge_index[0].astype(jnp.int32)
    dst = edge_index[1].astype(jnp.int32)
    x_pad = jnp.zeros((N_PAD, D), jnp.float32).at[:N_NODES].set(x)
    zeros_n = jnp.zeros((N_PAD,), jnp.float32)
    zeros_nd = jnp.zeros((N_PAD, D), jnp.float32)

    degp = _sc_degree(dst, zeros_n)
    y1, dinv = _tc_pre(x_pad, W1, degp)

    sp1 = _sc_conv(y1, src, dst, jnp.stack([y1, zeros_nd]))
    y2 = _tc_mid(sp1, dinv, b1, W2)

    sp2 = _sc_conv2(y2, src, dst, jnp.stack([y2, zeros_nd]))
    a, b = _tc_post(sp2, dinv, b2, fc1W, fc1b)

    e0 = _sc_edge_gather(a, b, src, dst)
    return _tc_mlp(e0, fc2W, fc2b, fc3W, fc3b)
